# Initial kernel scaffold; baseline (speedup 1.0000x reference)
import jax
import jax.numpy as jnp
from jax.experimental import pallas as pl


def kernel(x, Wr, br, W1, b1, W2, b2):
    raise NotImplementedError("write your pallas kernel here")


# speedup vs baseline: 3.7236x; 3.7236x over previous
import functools

import jax
import jax.numpy as jnp
from jax import lax
from jax.experimental import pallas as pl
from jax.experimental.pallas import tpu as pltpu
from jax.experimental.pallas import tpu_sc as plsc

S = 2048
D = 768
E = 8
DFF = 3072
TM = 256
PAD = S * 2 + E * TM
NT = PAD // TM

NW = 32
PAIRS = 2 * S
BP = PAIRS // NW
BT = S // NW


def _cumsum_excl(a):
    incl = a
    k = 1
    n = a.shape[0]
    while k < n:
        zero = jnp.zeros((k, a.shape[1]), a.dtype)
        incl = incl + jnp.concatenate([zero, incl[:-k]], axis=0)
        k *= 2
    return incl - a


def _router_meta_kernel(x_ref, wr_ref, br_ref, d0_ref, d1_ref, w_ref, te_ref):
    x = x_ref[...]
    logits = jnp.dot(x, wr_ref[...], preferred_element_type=jnp.float32)
    logits = logits + br_ref[...]
    ecols = lax.broadcasted_iota(jnp.int32, (S, E), 1)

    m0 = jnp.max(logits, axis=1, keepdims=True)
    a0 = jnp.min(jnp.where(logits == m0, ecols, E), axis=1, keepdims=True)
    rest = jnp.where(ecols == a0, -jnp.inf, logits)
    m1 = jnp.max(rest, axis=1, keepdims=True)
    a1 = jnp.min(jnp.where(rest == m1, ecols, E), axis=1, keepdims=True)

    e1 = jnp.exp(m1 - m0)
    w0 = 1.0 / (1.0 + e1)
    w1 = e1 / (1.0 + e1)

    oh0 = (ecols == a0).astype(jnp.float32)
    oh1 = (ecols == a1).astype(jnp.float32)
    c0x = _cumsum_excl(oh0)
    c1x = _cumsum_excl(oh1)
    cnt0 = jnp.sum(oh0, axis=0, keepdims=True)
    counts = cnt0 + jnp.sum(oh1, axis=0, keepdims=True)

    padded = jnp.ceil(counts / TM) * TM
    r8 = lax.broadcasted_iota(jnp.int32, (E, E), 0)
    c8 = lax.broadcasted_iota(jnp.int32, (E, E), 1)
    tri_incl = (r8 <= c8).astype(jnp.float32)
    incl = jnp.dot(padded, tri_incl, preferred_element_type=jnp.float32)
    offs = incl - padded

    r0 = jnp.sum(c0x * oh0, axis=1, keepdims=True)
    r1 = jnp.sum((cnt0 + c1x) * oh1, axis=1, keepdims=True)
    d0 = jnp.sum(offs * oh0, axis=1, keepdims=True) + r0
    d1 = jnp.sum(offs * oh1, axis=1, keepdims=True) + r1
    d0_ref[...] = d0.astype(jnp.int32)
    d1_ref[...] = d1.astype(jnp.int32)
    w_ref[...] = jnp.concatenate([w0, w1], axis=1)

    rs = lax.broadcasted_iota(jnp.int32, (NT, 1), 0).astype(jnp.float32) * TM
    te = jnp.sum((incl <= rs).astype(jnp.int32), axis=1, keepdims=True)
    te_ref[...] = jnp.minimum(te, E - 1)


def _dispatch_body(x_ref, dcat_ref, xg_ref, dest_v, rows_v, sem):
    wid = lax.axis_index("s") * 2 + lax.axis_index("c")
    base = wid * BP
    tbase = base - (base // S) * S
    pltpu.sync_copy(dcat_ref.at[pl.ds(base, BP)], dest_v)
    pltpu.sync_copy(x_ref.at[pl.ds(tbase, BP)], rows_v)
    pltpu.async_copy(rows_v, xg_ref.at[dest_v], sem).wait()


def _combine_body(y_ref, dcat_ref, yg_ref, idx_v, rows_v, sem):
    wid = lax.axis_index("s") * 2 + lax.axis_index("c")
    tbase = wid * BT
    for s in range(2):
        pltpu.sync_copy(dcat_ref.at[pl.ds(s * S + tbase, BT)], idx_v)
        pltpu.async_copy(y_ref.at[idx_v], rows_v, sem).wait()
        pltpu.sync_copy(rows_v, yg_ref.at[pl.ds(s * S + tbase, BT)])


def _ffn_kernel(te_ref, xg_ref, w1_ref, b1_ref, w2_ref, b2_ref, y_ref):
    del te_ref
    xg = xg_ref[...]
    h = jnp.dot(xg, w1_ref[0], preferred_element_type=jnp.float32) + b1_ref[0]
    h = 0.5 * h * (1.0 + lax.erf(h * 0.7071067811865476))
    y_ref[...] = (
        jnp.dot(h, w2_ref[0], preferred_element_type=jnp.float32) + b2_ref[0]
    )


def _wsum_kernel(y0_ref, y1_ref, w_ref, o_ref):
    w = w_ref[...]
    o_ref[...] = y0_ref[...] * w[:, 0:1] + y1_ref[...] * w[:, 1:2]


def kernel(x, Wr, br, W1, b1, W2, b2):
    Bs, Ss, Dd = x.shape
    x2 = x.reshape(S, D)

    d0, d1, w, te = pl.pallas_call(
        _router_meta_kernel,
        out_shape=[
            jax.ShapeDtypeStruct((S, 1), jnp.int32),
            jax.ShapeDtypeStruct((S, 1), jnp.int32),
            jax.ShapeDtypeStruct((S, 2), jnp.float32),
            jax.ShapeDtypeStruct((NT, 1), jnp.int32),
        ],
    )(x2, Wr, br.reshape(1, E))

    dcat = jnp.concatenate([d0.reshape(S), d1.reshape(S)])
    te_flat = te.reshape(NT)

    mesh = plsc.VectorSubcoreMesh(core_axis_name="c", subcore_axis_name="s")

    dispatch = functools.partial(
        pl.kernel,
        mesh=mesh,
        out_type=jax.ShapeDtypeStruct((PAD, D), jnp.float32),
        scratch_types=[
            pltpu.VMEM((BP,), jnp.int32),
            pltpu.VMEM((BP, D), jnp.float32),
            pltpu.SemaphoreType.DMA,
        ],
    )(_dispatch_body)
    xg = dispatch(x2, dcat)

    grid_spec = pltpu.PrefetchScalarGridSpec(
        num_scalar_prefetch=1,
        grid=(NT,),
        in_specs=[
            pl.BlockSpec((TM, D), lambda t, te: (t, 0)),
            pl.BlockSpec((1, D, DFF), lambda t, te: (te[t], 0, 0)),
            pl.BlockSpec((1, 1, DFF), lambda t, te: (te[t], 0, 0)),
            pl.BlockSpec((1, DFF, D), lambda t, te: (te[t], 0, 0)),
            pl.BlockSpec((1, 1, D), lambda t, te: (te[t], 0, 0)),
        ],
        out_specs=pl.BlockSpec((TM, D), lambda t, te: (t, 0)),
    )
    y = pl.pallas_call(
        _ffn_kernel,
        grid_spec=grid_spec,
        out_shape=jax.ShapeDtypeStruct((PAD, D), jnp.float32),
    )(te_flat, xg, W1, b1.reshape(E, 1, DFF), W2, b2.reshape(E, 1, D))

    combine = functools.partial(
        pl.kernel,
        mesh=mesh,
        out_type=jax.ShapeDtypeStruct((PAIRS, D), jnp.float32),
        scratch_types=[
            pltpu.VMEM((BT,), jnp.int32),
            pltpu.VMEM((BT, D), jnp.float32),
            pltpu.SemaphoreType.DMA,
        ],
    )(_combine_body)
    yg = combine(y, dcat)

    n2 = S // TM
    out = pl.pallas_call(
        _wsum_kernel,
        grid=(n2,),
        in_specs=[
            pl.BlockSpec((TM, D), lambda i: (i, 0)),
            pl.BlockSpec((TM, D), lambda i: (i + n2, 0)),
            pl.BlockSpec((TM, 2), lambda i: (i, 0)),
        ],
        out_specs=pl.BlockSpec((TM, D), lambda i: (i, 0)),
        out_shape=jax.ShapeDtypeStruct((S, D), jnp.float32),
    )(yg, yg, w)

    return out.reshape(Bs, Ss, Dd)
